# baseline (device time: 37486 ns/iter reference)
import jax
import jax.numpy as jnp
from jax import lax
from jax.experimental import pallas as pl
from jax.experimental.pallas import tpu as pltpu

M = 1024
N = 1024
K = 4096
Q = 512
HB = Q // 2
NC = 4
CC = Q // NC
CDT = jnp.bfloat16


def kernel(dy, W):
    def body(dy_ref, w_ref, out_ref,
             dyb_ref, part_ref, yrecv_ref,
             own_ref, xrecv_ref, zrecv_ref, dga_ref, dgb_ref,
             ldma_sems,
             y_send, y_recv, x_send, x_recv,
             za_send, za_recv, zb_send, zb_recv, xb_send, xb_recv):
        mx = lax.axis_index("x")
        my = lax.axis_index("y")
        mz = lax.axis_index("z")
        r0 = mz * Q
        rz = (1 - mz) * Q
        c0 = mx * Q
        cx = (1 - mx) * Q

        dy_cp = pltpu.make_async_copy(
            dy_ref.at[pl.ds(r0, Q), :], dyb_ref, ldma_sems.at[0])
        dy_cp.start()

        barrier = pltpu.get_barrier_semaphore()
        for nbr in ((1 - mx, my, mz), (mx, 1 - my, mz), (mx, my, 1 - mz)):
            pl.semaphore_signal(
                barrier, inc=1, device_id=nbr,
                device_id_type=pl.DeviceIdType.MESH,
            )
        pl.semaphore_wait(barrier, 3)

        def rdma(src, dst, ssem, rsem, dev):
            return pltpu.make_async_remote_copy(
                src_ref=src, dst_ref=dst, send_sem=ssem, recv_sem=rsem,
                device_id=dev, device_id_type=pl.DeviceIdType.MESH)

        y_nbr = (mx, 1 - my, mz)
        x_nbr = (1 - mx, my, mz)
        z_nbr = (mx, my, 1 - mz)

        y_rdmas, x_rdmas, za_rdmas, zb_rdmas, xb_rdmas = [], [], [], [], []
        for c in range(NC):
            y_rdmas.append(rdma(part_ref.at[c], yrecv_ref.at[c],
                                y_send.at[c], y_recv.at[c], y_nbr))
            x_rdmas.append(rdma(own_ref.at[c], xrecv_ref.at[c],
                                x_send.at[c], x_recv.at[c], x_nbr))
            za_rdmas.append(rdma(own_ref.at[c], zrecv_ref.at[c],
                                 za_send.at[c], za_recv.at[c], z_nbr))
            zb_rdmas.append(rdma(xrecv_ref.at[c, pl.ds(0, HB), :],
                                 dga_ref.at[c],
                                 zb_send.at[c], zb_recv.at[c], z_nbr))
            xb_rdmas.append(rdma(zrecv_ref.at[c, pl.ds(HB, HB), :],
                                 dgb_ref.at[c],
                                 xb_send.at[c], xb_recv.at[c], x_nbr))

        def drain_y(c):
            y_rdmas[c].wait_recv()
            sum32 = (part_ref[c].astype(jnp.float32)
                     + yrecv_ref[c].astype(jnp.float32))
            own_ref[c] = sum32.astype(CDT)
            x_rdmas[c].start()
            za_rdmas[c].start()
            out_ref[pl.ds(r0, Q), pl.ds(c0 + c * CC, CC)] = sum32

        dy_cp.wait()
        for c in range(NC):
            part_ref[c] = lax.dot_general(
                dyb_ref[...], w_ref[pl.ds(c0 + c * CC, CC), :],
                dimension_numbers=(((1,), (1,)), ((), ())),
                preferred_element_type=jnp.float32,
            ).astype(CDT)
            y_rdmas[c].start()
            if c >= 1:
                drain_y(c - 1)
        drain_y(NC - 1)

        for c in range(NC):
            x_rdmas[c].wait_recv()
            zb_rdmas[c].start()
            out_ref[pl.ds(r0, Q), pl.ds(cx + c * CC, CC)] = (
                xrecv_ref[c].astype(jnp.float32))
            za_rdmas[c].wait_recv()
            xb_rdmas[c].start()
            out_ref[pl.ds(rz, Q), pl.ds(c0 + c * CC, CC)] = (
                zrecv_ref[c].astype(jnp.float32))

        for c in range(NC):
            zb_rdmas[c].wait_recv()
            out_ref[pl.ds(rz, HB), pl.ds(cx + c * CC, CC)] = (
                dga_ref[c].astype(jnp.float32))
            xb_rdmas[c].wait_recv()
            out_ref[pl.ds(rz + HB, HB), pl.ds(cx + c * CC, CC)] = (
                dgb_ref[c].astype(jnp.float32))

        for c in range(NC):
            y_rdmas[c].wait_send()
            x_rdmas[c].wait_send()
            za_rdmas[c].wait_send()
            zb_rdmas[c].wait_send()
            xb_rdmas[c].wait_send()

    return pl.pallas_call(
        body,
        out_shape=jax.ShapeDtypeStruct((M, N), jnp.float32),
        in_specs=[
            pl.BlockSpec(memory_space=pltpu.MemorySpace.HBM),
            pl.BlockSpec(memory_space=pltpu.VMEM),
        ],
        out_specs=pl.BlockSpec(memory_space=pltpu.VMEM),
        scratch_shapes=[
            pltpu.VMEM((Q, K), jnp.float32),
            pltpu.VMEM((NC, Q, CC), CDT),
            pltpu.VMEM((NC, Q, CC), CDT),
            pltpu.VMEM((NC, Q, CC), CDT),
            pltpu.VMEM((NC, Q, CC), CDT),
            pltpu.VMEM((NC, Q, CC), CDT),
            pltpu.VMEM((NC, HB, CC), CDT),
            pltpu.VMEM((NC, HB, CC), CDT),
            pltpu.SemaphoreType.DMA((1,)),
            pltpu.SemaphoreType.DMA((NC,)),
            pltpu.SemaphoreType.DMA((NC,)),
            pltpu.SemaphoreType.DMA((NC,)),
            pltpu.SemaphoreType.DMA((NC,)),
            pltpu.SemaphoreType.DMA((NC,)),
            pltpu.SemaphoreType.DMA((NC,)),
            pltpu.SemaphoreType.DMA((NC,)),
            pltpu.SemaphoreType.DMA((NC,)),
            pltpu.SemaphoreType.DMA((NC,)),
            pltpu.SemaphoreType.DMA((NC,)),
        ],
        compiler_params=pltpu.CompilerParams(collective_id=0),
    )(dy, W)


# device time: 31994 ns/iter; 1.1717x vs baseline; 1.1717x over previous
import jax
import jax.numpy as jnp
from jax import lax
from jax.experimental import pallas as pl
from jax.experimental.pallas import tpu as pltpu

M = 1024
N = 1024
K = 4096
Q = 512
HB = Q // 2
NC = 4
CC = Q // NC
CDT = jnp.bfloat16


def kernel(dy, W):
    def body(dy_ref, w_ref, out_ref,
             dyb_ref, wbuf_ref, part_ref, yrecv_ref,
             own_ref, xrecv_ref, zrecv_ref, dga_ref, dgb_ref,
             ldma_sems,
             y_send, y_recv, x_send, x_recv,
             za_send, za_recv, zb_send, zb_recv, xb_send, xb_recv):
        mx = lax.axis_index("x")
        my = lax.axis_index("y")
        mz = lax.axis_index("z")
        r0 = mz * Q
        rz = (1 - mz) * Q
        c0 = mx * Q
        cx = (1 - mx) * Q

        dy_cp = pltpu.make_async_copy(
            dy_ref.at[pl.ds(r0, Q), :], dyb_ref, ldma_sems.at[2])
        dy_cp.start()
        w_cps = [
            pltpu.make_async_copy(
                w_ref.at[pl.ds(c0 + c * CC, CC), :], wbuf_ref.at[c % 2],
                ldma_sems.at[c % 2])
            for c in range(NC)
        ]
        w_cps[0].start()

        barrier = pltpu.get_barrier_semaphore()
        for nbr in ((1 - mx, my, mz), (mx, 1 - my, mz), (mx, my, 1 - mz)):
            pl.semaphore_signal(
                barrier, inc=1, device_id=nbr,
                device_id_type=pl.DeviceIdType.MESH,
            )
        pl.semaphore_wait(barrier, 3)

        def rdma(src, dst, ssem, rsem, dev):
            return pltpu.make_async_remote_copy(
                src_ref=src, dst_ref=dst, send_sem=ssem, recv_sem=rsem,
                device_id=dev, device_id_type=pl.DeviceIdType.MESH)

        y_nbr = (mx, 1 - my, mz)
        x_nbr = (1 - mx, my, mz)
        z_nbr = (mx, my, 1 - mz)

        y_rdmas, x_rdmas, za_rdmas, zb_rdmas, xb_rdmas = [], [], [], [], []
        for c in range(NC):
            y_rdmas.append(rdma(part_ref.at[c], yrecv_ref.at[c],
                                y_send.at[c], y_recv.at[c], y_nbr))
            x_rdmas.append(rdma(own_ref.at[c], xrecv_ref.at[c],
                                x_send.at[c], x_recv.at[c], x_nbr))
            za_rdmas.append(rdma(own_ref.at[c], zrecv_ref.at[c],
                                 za_send.at[c], za_recv.at[c], z_nbr))
            zb_rdmas.append(rdma(xrecv_ref.at[c, pl.ds(0, HB), :],
                                 dga_ref.at[c],
                                 zb_send.at[c], zb_recv.at[c], z_nbr))
            xb_rdmas.append(rdma(zrecv_ref.at[c, pl.ds(HB, HB), :],
                                 dgb_ref.at[c],
                                 xb_send.at[c], xb_recv.at[c], x_nbr))

        def drain_y(c):
            y_rdmas[c].wait_recv()
            sum32 = (part_ref[c].astype(jnp.float32)
                     + yrecv_ref[c].astype(jnp.float32))
            own_ref[c] = sum32.astype(CDT)
            x_rdmas[c].start()
            za_rdmas[c].start()
            out_ref[pl.ds(r0, Q), pl.ds(c0 + c * CC, CC)] = sum32

        dy_cp.wait()
        for c in range(NC):
            w_cps[c].wait()
            if c + 1 < NC:
                w_cps[c + 1].start()
            part_ref[c] = lax.dot_general(
                dyb_ref[...], wbuf_ref[c % 2],
                dimension_numbers=(((1,), (1,)), ((), ())),
                preferred_element_type=jnp.float32,
            ).astype(CDT)
            y_rdmas[c].start()
            if c >= 1:
                drain_y(c - 1)
        drain_y(NC - 1)

        for c in range(NC):
            x_rdmas[c].wait_recv()
            zb_rdmas[c].start()
            out_ref[pl.ds(r0, Q), pl.ds(cx + c * CC, CC)] = (
                xrecv_ref[c].astype(jnp.float32))
            za_rdmas[c].wait_recv()
            xb_rdmas[c].start()
            out_ref[pl.ds(rz, Q), pl.ds(c0 + c * CC, CC)] = (
                zrecv_ref[c].astype(jnp.float32))

        for c in range(NC):
            zb_rdmas[c].wait_recv()
            out_ref[pl.ds(rz, HB), pl.ds(cx + c * CC, CC)] = (
                dga_ref[c].astype(jnp.float32))
            xb_rdmas[c].wait_recv()
            out_ref[pl.ds(rz + HB, HB), pl.ds(cx + c * CC, CC)] = (
                dgb_ref[c].astype(jnp.float32))

        for c in range(NC):
            y_rdmas[c].wait_send()
            x_rdmas[c].wait_send()
            za_rdmas[c].wait_send()
            zb_rdmas[c].wait_send()
            xb_rdmas[c].wait_send()

    return pl.pallas_call(
        body,
        out_shape=jax.ShapeDtypeStruct((M, N), jnp.float32),
        in_specs=[
            pl.BlockSpec(memory_space=pltpu.MemorySpace.HBM),
            pl.BlockSpec(memory_space=pltpu.MemorySpace.HBM),
        ],
        out_specs=pl.BlockSpec(memory_space=pltpu.VMEM),
        scratch_shapes=[
            pltpu.VMEM((Q, K), jnp.float32),
            pltpu.VMEM((2, CC, K), jnp.float32),
            pltpu.VMEM((NC, Q, CC), CDT),
            pltpu.VMEM((NC, Q, CC), CDT),
            pltpu.VMEM((NC, Q, CC), CDT),
            pltpu.VMEM((NC, Q, CC), CDT),
            pltpu.VMEM((NC, Q, CC), CDT),
            pltpu.VMEM((NC, HB, CC), CDT),
            pltpu.VMEM((NC, HB, CC), CDT),
            pltpu.SemaphoreType.DMA((3,)),
            pltpu.SemaphoreType.DMA((NC,)),
            pltpu.SemaphoreType.DMA((NC,)),
            pltpu.SemaphoreType.DMA((NC,)),
            pltpu.SemaphoreType.DMA((NC,)),
            pltpu.SemaphoreType.DMA((NC,)),
            pltpu.SemaphoreType.DMA((NC,)),
            pltpu.SemaphoreType.DMA((NC,)),
            pltpu.SemaphoreType.DMA((NC,)),
            pltpu.SemaphoreType.DMA((NC,)),
            pltpu.SemaphoreType.DMA((NC,)),
        ],
        compiler_params=pltpu.CompilerParams(collective_id=0),
    )(dy, W)
